# baseline (speedup 1.0000x reference)
import jax
import jax.numpy as jnp
from jax.experimental import pallas as pl
from jax.experimental.pallas import tpu as pltpu


def _lstm_ae_kernel(x_ref, wih_e_hbm, b_e_ref, whh_e_hbm,
                    wih_d_hbm, whh_d_hbm, b_d_ref, out_ref,
                    wih_e_v, whh_e_v, wih_d_v, whh_d_v, sems):
    Bt, T, I = x_ref.shape
    H = whh_e_v.shape[0]
    f32 = jnp.float32

    cp_wih_e = pltpu.make_async_copy(wih_e_hbm, wih_e_v, sems.at[0])
    cp_whh_e = pltpu.make_async_copy(whh_e_hbm, whh_e_v, sems.at[1])
    cp_wih_d = pltpu.make_async_copy(wih_d_hbm, wih_d_v, sems.at[2])
    cp_whh_d = pltpu.make_async_copy(whh_d_hbm, whh_d_v, sems.at[3])
    cp_wih_e.start()
    cp_whh_e.start()
    cp_wih_d.start()
    cp_whh_d.start()

    cp_wih_e.wait()
    x = x_ref[...]
    xw = jnp.dot(x.reshape(Bt * T, I), wih_e_v[...],
                 preferred_element_type=f32) + b_e_ref[...]
    xw = xw.reshape(Bt, T, 4 * H)

    cp_whh_e.wait()
    whh_e = whh_e_v[...]

    h = jnp.zeros((Bt, H), f32)
    c = jnp.zeros((Bt, H), f32)
    for t in range(T):
        gates = xw[:, t, :] + jnp.dot(h, whh_e, preferred_element_type=f32)
        sig = jnp.tanh(gates[:, :3 * H] * 0.5) * 0.5 + 0.5
        g_g = jnp.tanh(gates[:, 3 * H:])
        i_g = sig[:, 0 * H:1 * H]
        f_g = sig[:, 1 * H:2 * H]
        o_g = sig[:, 2 * H:3 * H]
        c = f_g * c + i_g * g_g
        h = o_g * jnp.tanh(c)

    cp_wih_d.wait()
    xw_d = jnp.dot(h, wih_d_v[...], preferred_element_type=f32) + b_d_ref[...]

    cp_whh_d.wait()
    whh_d = whh_d_v[...]

    hd = jnp.zeros((Bt, I), f32)
    cd = jnp.zeros((Bt, I), f32)
    for t in range(T):
        gates = xw_d + jnp.dot(hd, whh_d, preferred_element_type=f32)
        sig = jnp.tanh(gates[:, :3 * I] * 0.5) * 0.5 + 0.5
        g_g = jnp.tanh(gates[:, 3 * I:])
        i_g = sig[:, 0 * I:1 * I]
        f_g = sig[:, 1 * I:2 * I]
        o_g = sig[:, 2 * I:3 * I]
        cd = f_g * cd + i_g * g_g
        hd = o_g * jnp.tanh(cd)
        out_ref[:, t * I:(t + 1) * I] = hd


@jax.jit
def _forward(x, enc_wih_t, enc_b, enc_whh_t, dec_wih_t, dec_whh_t, dec_b):
    B, T, I = x.shape
    H = enc_whh_t.shape[0]
    f32 = jnp.float32

    bt = B // 2 if (B % 16 == 0) else B
    grid = (B // bt,)
    anyspace = pl.BlockSpec(memory_space=pl.ANY)

    out_flat = pl.pallas_call(
        _lstm_ae_kernel,
        out_shape=jax.ShapeDtypeStruct((B, T * I), f32),
        grid=grid,
        in_specs=[
            pl.BlockSpec((bt, T, I), lambda b: (b, 0, 0)),
            anyspace,
            pl.BlockSpec((1, 4 * H), lambda b: (0, 0)),
            anyspace,
            anyspace,
            anyspace,
            pl.BlockSpec((1, 4 * I), lambda b: (0, 0)),
        ],
        out_specs=pl.BlockSpec((bt, T * I), lambda b: (b, 0)),
        scratch_shapes=[
            pltpu.VMEM((I, 4 * H), f32),
            pltpu.VMEM((H, 4 * H), f32),
            pltpu.VMEM((H, 4 * I), f32),
            pltpu.VMEM((I, 4 * I), f32),
            pltpu.SemaphoreType.DMA((4,)),
        ],
        compiler_params=pltpu.CompilerParams(
            dimension_semantics=("parallel",),
            vmem_limit_bytes=64 * 1024 * 1024),
    )(x, enc_wih_t, enc_b, enc_whh_t, dec_wih_t, dec_whh_t, dec_b)

    return out_flat.reshape(B, T, I)


def kernel(x, enc_wih_t, enc_b, enc_whh_t, dec_wih_t, dec_whh_t, dec_b):
    return _forward(x, enc_wih_t, enc_b, enc_whh_t, dec_wih_t,
                    dec_whh_t, dec_b)
